# baseline (speedup 1.0000x reference)
import functools

import jax
import jax.numpy as jnp
from jax.experimental import pallas as pl
from jax.experimental.pallas import tpu as pltpu
from jax.experimental.pallas import tpu_sc as plsc

_W = 128


def _sc_gather(table, idx2d, n):
    d = table.shape[-1]
    g = idx2d.shape[0]
    mesh = plsc.VectorSubcoreMesh(core_axis_name="c", subcore_axis_name="s")

    @functools.partial(
        pl.kernel,
        out_type=jax.ShapeDtypeStruct((n, d), jnp.float32),
        mesh=mesh,
    )
    def gather_kernel(tok_hbm, idx_hbm, out_hbm):
        def body(idx_vmem, out_vmem):
            pltpu.sync_copy(tok_hbm.at[idx_vmem.at[0]], out_vmem)

        pltpu.emit_pipeline(
            body,
            grid=(g,),
            in_specs=[pl.BlockSpec((1, _W), index_map=lambda i: (i, 0))],
            out_specs=[pl.BlockSpec((_W, d), index_map=lambda i: (i, 0))],
            core_axis_name=("c", "s"),
            dimension_semantics=(pltpu.PARALLEL,),
        )(idx_hbm, out_hbm)

    return gather_kernel(table, idx2d)


def _tc_layernorm(tok_packed, parity, seg, pos_slice, seg_embed, gamma, beta):
    b, l = seg.shape
    d = pos_slice.shape[-1]
    bb = 64

    def ln_kernel(tok_ref, par_ref, seg_ref, pos_ref, sege_ref, g_ref, b_ref,
                  o_ref):
        packed = tok_ref[...]
        lo = packed[..., :d]
        hi = packed[..., d:]
        par = par_ref[...].astype(jnp.float32)[..., None]
        tok = lo + par * (hi - lo)
        segf = seg_ref[...].astype(jnp.float32)[..., None]
        se0 = sege_ref[0, :]
        se1 = sege_ref[1, :]
        emb = tok + pos_ref[...][None, :, :] + (se0 + segf * (se1 - se0))
        mean = jnp.mean(emb, axis=-1, keepdims=True)
        cent = emb - mean
        var = jnp.mean(cent * cent, axis=-1, keepdims=True)
        o_ref[...] = cent * jax.lax.rsqrt(var + 1e-5) * g_ref[0, :] + b_ref[0, :]

    return pl.pallas_call(
        ln_kernel,
        grid=(b // bb,),
        in_specs=[
            pl.BlockSpec((bb, l, 2 * d), lambda i: (i, 0, 0)),
            pl.BlockSpec((bb, l), lambda i: (i, 0)),
            pl.BlockSpec((bb, l), lambda i: (i, 0)),
            pl.BlockSpec((l, d), lambda i: (0, 0)),
            pl.BlockSpec((2, d), lambda i: (0, 0)),
            pl.BlockSpec((1, d), lambda i: (0, 0)),
            pl.BlockSpec((1, d), lambda i: (0, 0)),
        ],
        out_specs=pl.BlockSpec((bb, l, d), lambda i: (i, 0, 0)),
        out_shape=jax.ShapeDtypeStruct((b, l, d), jnp.float32),
    )(
        tok_packed.reshape(b, l, 2 * d),
        parity,
        seg,
        pos_slice,
        seg_embed,
        gamma.reshape(1, d),
        beta.reshape(1, d),
    )


def kernel(x, seg, tok_embed, pos_embed, seg_embed, gamma, beta):
    b, l = x.shape
    v, d = tok_embed.shape
    n = b * l
    xi = x.astype(jnp.int32)
    parity = xi & 1
    idx2d = (xi >> 1).reshape(n // _W, _W)
    table128 = tok_embed.reshape(v // 2, 2 * d)
    rows = _sc_gather(table128, idx2d, n)
    pos_slice = jax.lax.slice(pos_embed, (0, 0), (l, d))
    return _tc_layernorm(rows, parity, seg.astype(jnp.int32), pos_slice,
                         seg_embed, gamma, beta)


# speedup vs baseline: 1.0381x; 1.0381x over previous
import functools

import jax
import jax.numpy as jnp
from jax.experimental import pallas as pl
from jax.experimental.pallas import tpu as pltpu
from jax.experimental.pallas import tpu_sc as plsc

_W = 128


def _sc_gather(table, idx2d, n):
    d = table.shape[-1]
    g = idx2d.shape[0]
    mesh = plsc.VectorSubcoreMesh(core_axis_name="c", subcore_axis_name="s")

    @functools.partial(
        pl.kernel,
        out_type=jax.ShapeDtypeStruct((n, d), jnp.float32),
        mesh=mesh,
        compiler_params=pltpu.CompilerParams(use_tc_tiling_on_sc=False),
    )
    def gather_kernel(tok_hbm, idx_hbm, out_hbm):
        def body(idx_vmem, out_vmem):
            pltpu.sync_copy(tok_hbm.at[idx_vmem.at[0]], out_vmem)

        pltpu.emit_pipeline(
            body,
            grid=(g,),
            in_specs=[pl.BlockSpec((1, _W), index_map=lambda i: (i, 0))],
            out_specs=[pl.BlockSpec((_W, d), index_map=lambda i: (i, 0))],
            core_axis_name=("c", "s"),
            dimension_semantics=(pltpu.PARALLEL,),
        )(idx_hbm, out_hbm)

    return gather_kernel(table, idx2d)


def _tc_layernorm(tok_rows, seg, pos_slice, seg_embed, gamma, beta):
    b, l = seg.shape
    d = pos_slice.shape[-1]
    bb = 64

    def ln_kernel(tok_ref, seg_ref, pos_ref, sege_ref, g_ref, b_ref, o_ref):
        tok = tok_ref[...]
        segf = seg_ref[...].astype(jnp.float32)[..., None]
        se0 = sege_ref[0, :]
        se1 = sege_ref[1, :]
        emb = tok + pos_ref[...][None, :, :] + (se0 + segf * (se1 - se0))
        mean = jnp.mean(emb, axis=-1, keepdims=True)
        cent = emb - mean
        var = jnp.mean(cent * cent, axis=-1, keepdims=True)
        o_ref[...] = cent * jax.lax.rsqrt(var + 1e-5) * g_ref[0, :] + b_ref[0, :]

    return pl.pallas_call(
        ln_kernel,
        grid=(b // bb,),
        in_specs=[
            pl.BlockSpec((bb, l, d), lambda i: (i, 0, 0)),
            pl.BlockSpec((bb, l), lambda i: (i, 0)),
            pl.BlockSpec((l, d), lambda i: (0, 0)),
            pl.BlockSpec((2, d), lambda i: (0, 0)),
            pl.BlockSpec((1, d), lambda i: (0, 0)),
            pl.BlockSpec((1, d), lambda i: (0, 0)),
        ],
        out_specs=pl.BlockSpec((bb, l, d), lambda i: (i, 0, 0)),
        out_shape=jax.ShapeDtypeStruct((b, l, d), jnp.float32),
    )(
        tok_rows.reshape(b, l, d),
        seg,
        pos_slice,
        seg_embed,
        gamma.reshape(1, d),
        beta.reshape(1, d),
    )


def kernel(x, seg, tok_embed, pos_embed, seg_embed, gamma, beta):
    b, l = x.shape
    d = tok_embed.shape[1]
    n = b * l
    idx2d = x.astype(jnp.int32).reshape(n // _W, _W)
    rows = _sc_gather(tok_embed, idx2d, n)
    pos_slice = jax.lax.slice(pos_embed, (0, 0), (l, d))
    return _tc_layernorm(rows, seg.astype(jnp.int32), pos_slice, seg_embed,
                         gamma, beta)
